# Initial kernel scaffold; baseline (speedup 1.0000x reference)
import jax
import jax.numpy as jnp
from jax.experimental import pallas as pl


def kernel(x, edge_index, W1_rel, b1, W1_root, Wl1, bl1, W2_rel, b2, W2_root, Wl2, bl2):
    raise NotImplementedError("write your pallas kernel here")


# speedup vs baseline: 7.0073x; 7.0073x over previous
import functools

import jax
import jax.numpy as jnp
from jax import lax
from jax.experimental import pallas as pl
from jax.experimental.pallas import tpu as pltpu
from jax.experimental.pallas import tpu_sc as plsc

N_NODES = 10000
N_EDGES = 320000
N_PAD = 10240
NC, NS = 2, 16
NW = NC * NS
CHUNK = 128
CPW = 80
N_CHUNKS = NW * CPW
ROW_BLK = 1280


@functools.lru_cache(maxsize=None)
def _make_segsum(d):
    rpt = N_PAD // NS
    mesh = plsc.VectorSubcoreMesh(core_axis_name="c", subcore_axis_name="s")

    @functools.partial(
        pl.kernel,
        out_type=jax.ShapeDtypeStruct((NC, N_PAD, d), jnp.float32),
        mesh=mesh,
        scratch_types=[
            pltpu.VMEM((CPW, CHUNK), jnp.int32),
            pltpu.VMEM((CPW, CHUNK), jnp.int32),
            pltpu.VMEM((CHUNK, d), jnp.float32),
            pltpu.VMEM_SHARED((N_PAD, d), jnp.float32),
            pltpu.SemaphoreType.DMA,
        ],
        compiler_params=pltpu.CompilerParams(use_tc_tiling_on_sc=False),
    )
    def segsum(table, srcc, dstc, zrows, out, src_v, dst_v, rows_v, acc, sem):
        c = lax.axis_index("c")
        s = lax.axis_index("s")
        wid = c * NS + s
        pltpu.sync_copy(zrows.at[pl.ds(s * rpt, rpt)],
                        acc.at[pl.ds(s * rpt, rpt)])
        pltpu.sync_copy(srcc.at[pl.ds(wid * CPW, CPW)], src_v)
        pltpu.sync_copy(dstc.at[pl.ds(wid * CPW, CPW)], dst_v)
        plsc.subcore_barrier()

        def body(j, carry):
            pltpu.async_copy(table.at[src_v.at[j]], rows_v, sem).wait()
            pltpu.sync_copy(rows_v, acc.at[dst_v.at[j]], add=True)
            return carry

        lax.fori_loop(0, CPW, body, 0)
        plsc.subcore_barrier()
        pltpu.sync_copy(acc.at[pl.ds(s * rpt, rpt)],
                        out.at[c].at[pl.ds(s * rpt, rpt)])

    return segsum


def _segment_sum_sc(table, srcc, dstc, zrows):
    return _make_segsum(table.shape[1])(table, srcc, dstc, zrows)


def _proj_body(x_ref, w_ref, o_ref):
    o_ref[...] = jnp.dot(x_ref[...], w_ref[...],
                         preferred_element_type=jnp.float32)


def _proj(xp, w):
    n, k = xp.shape
    m = w.shape[1]
    return pl.pallas_call(
        _proj_body,
        grid=(n // ROW_BLK,),
        in_specs=[pl.BlockSpec((ROW_BLK, k), lambda i: (i, 0)),
                  pl.BlockSpec((k, m), lambda i: (0, 0))],
        out_specs=pl.BlockSpec((ROW_BLK, m), lambda i: (i, 0)),
        out_shape=jax.ShapeDtypeStruct((n, m), jnp.float32),
    )(xp, w)


def _mid_body(a0_ref, a1_ref, x_ref, wroot_ref, b1_ref, wl1_ref, bl1_ref,
              w2rel_ref, w2root_ref, b2_ref, p2_ref, r2_ref):
    h = (a0_ref[...] + a1_ref[...] + b1_ref[...]
         + jnp.dot(x_ref[...], wroot_ref[...],
                   preferred_element_type=jnp.float32))
    t = jax.nn.relu(jnp.dot(h, wl1_ref[...],
                            preferred_element_type=jnp.float32) + bl1_ref[...])
    p2_ref[...] = jnp.dot(t, w2rel_ref[...],
                          preferred_element_type=jnp.float32)
    r2_ref[...] = (jnp.dot(t, w2root_ref[...],
                           preferred_element_type=jnp.float32) + b2_ref[...])


def _mid(a0, a1, xp, w1_root, b1, wl1, bl1, w2_rel, w2_root, b2):
    n = xp.shape[0]
    full = lambda shape: pl.BlockSpec(shape, lambda i: (0, 0))
    row = lambda m: pl.BlockSpec((ROW_BLK, m), lambda i: (i, 0))
    return pl.pallas_call(
        _mid_body,
        grid=(n // ROW_BLK,),
        in_specs=[row(64), row(64), row(128), full((128, 64)), full((1, 64)),
                  full((64, 32)), full((1, 32)), full((32, 16)),
                  full((32, 16)), full((1, 16))],
        out_specs=[row(16), row(16)],
        out_shape=[jax.ShapeDtypeStruct((n, 16), jnp.float32),
                   jax.ShapeDtypeStruct((n, 16), jnp.float32)],
    )(a0, a1, xp, w1_root, b1.reshape(1, 64), wl1, bl1.reshape(1, 32),
      w2_rel, w2_root, b2.reshape(1, 16))


def _final_body(a0_ref, a1_ref, r2_ref, wl2_ref, bl2_ref, o_ref):
    h2 = a0_ref[...] + a1_ref[...] + r2_ref[...]
    logits = jnp.dot(h2, wl2_ref[...],
                     preferred_element_type=jnp.float32) + bl2_ref[...]
    m = jnp.max(logits, axis=1, keepdims=True)
    sh = logits - m
    lse = jnp.log(jnp.sum(jnp.exp(sh), axis=1, keepdims=True))
    o_ref[...] = sh - lse


def _final(a0, a1, r2, wl2, bl2):
    n = a0.shape[0]
    ncls = wl2.shape[1]
    full = lambda shape: pl.BlockSpec(shape, lambda i: (0, 0))
    row = lambda m: pl.BlockSpec((ROW_BLK, m), lambda i: (i, 0))
    return pl.pallas_call(
        _final_body,
        grid=(n // ROW_BLK,),
        in_specs=[row(16), row(16), row(16), full((16, ncls)),
                  full((1, ncls))],
        out_specs=row(ncls),
        out_shape=jax.ShapeDtypeStruct((n, ncls), jnp.float32),
    )(a0, a1, r2, wl2, bl2.reshape(1, ncls))


def kernel(x, edge_index, W1_rel, b1, W1_root, Wl1, bl1, W2_rel, b2, W2_root,
           Wl2, bl2):
    n = x.shape[0]
    e = edge_index.shape[1]
    xp = jnp.pad(x, ((0, N_PAD - n), (0, 0)))
    src = edge_index[0].astype(jnp.int32)
    dst = edge_index[1].astype(jnp.int32)
    pad_e = N_CHUNKS * CHUNK - e
    srcc = jnp.concatenate([src, jnp.full((pad_e,), n, jnp.int32)])
    srcc = srcc.reshape(N_CHUNKS, CHUNK)
    dstc = jnp.concatenate([dst, jnp.full((pad_e,), n, jnp.int32)])
    dstc = dstc.reshape(N_CHUNKS, CHUNK)
    z64 = jnp.zeros((N_PAD, 64), jnp.float32)
    z16 = jnp.zeros((N_PAD, 16), jnp.float32)

    p1 = _proj(xp, W1_rel)
    agg1 = _segment_sum_sc(p1, srcc, dstc, z64)
    p2, r2 = _mid(agg1[0], agg1[1], xp, W1_root, b1, Wl1, bl1,
                  W2_rel, W2_root, b2)
    agg2 = _segment_sum_sc(p2, srcc, dstc, z16)
    out = _final(agg2[0], agg2[1], r2, Wl2, bl2)
    return out[:n]


# speedup vs baseline: 8.7191x; 1.2443x over previous
import functools

import jax
import jax.numpy as jnp
from jax import lax
from jax.experimental import pallas as pl
from jax.experimental.pallas import tpu as pltpu
from jax.experimental.pallas import tpu_sc as plsc

N_NODES = 10000
N_EDGES = 320000
N_PAD = 10240
NC, NS = 2, 16
NW = NC * NS
CHUNK = 128
CPW = 80
N_CHUNKS = NW * CPW
ROW_BLK = 1280


GDEPTH = 8


@functools.lru_cache(maxsize=None)
def _make_segsum(d):
    rpt = N_PAD // NS
    mesh = plsc.VectorSubcoreMesh(core_axis_name="c", subcore_axis_name="s")

    @functools.partial(
        pl.kernel,
        out_type=jax.ShapeDtypeStruct((NC, N_PAD, d), jnp.float32),
        mesh=mesh,
        scratch_types=[
            pltpu.VMEM((CPW, CHUNK), jnp.int32),
            pltpu.VMEM((CPW, CHUNK), jnp.int32),
            pltpu.VMEM((GDEPTH, CHUNK, d), jnp.float32),
            pltpu.VMEM_SHARED((N_PAD, d), jnp.float32),
            pltpu.SemaphoreType.DMA((GDEPTH,)),
            pltpu.SemaphoreType.DMA,
        ],
        compiler_params=pltpu.CompilerParams(use_tc_tiling_on_sc=False),
    )
    def segsum(table, srcc, dstc, zrows, out, src_v, dst_v, rows_v, acc,
               gsem, sem):
        c = lax.axis_index("c")
        s = lax.axis_index("s")
        wid = c * NS + s
        cz = pltpu.async_copy(zrows.at[pl.ds(s * rpt, rpt)],
                              acc.at[pl.ds(s * rpt, rpt)], sem)
        ci = pltpu.async_copy(srcc.at[pl.ds(wid * CPW, CPW)], src_v, sem)
        cj = pltpu.async_copy(dstc.at[pl.ds(wid * CPW, CPW)], dst_v, sem)
        cz.wait()
        ci.wait()
        cj.wait()
        plsc.subcore_barrier()

        def gather(cc, b):
            pltpu.async_copy(table.at[src_v.at[cc]], rows_v.at[b], gsem.at[b])

        def gather_wait(cc, b):
            pltpu.make_async_copy(table.at[src_v.at[cc]], rows_v.at[b],
                                  gsem.at[b]).wait()

        for b in range(GDEPTH):
            gather(b, b)

        def body(i, carry):
            j = i * GDEPTH
            for b in range(GDEPTH):
                cc = j + b
                gather_wait(cc, b)
                pltpu.sync_copy(rows_v.at[b], acc.at[dst_v.at[cc]], add=True)
                nxt = cc + GDEPTH

                @pl.when(nxt < CPW)
                def _():
                    gather(nxt, b)
            return carry

        lax.fori_loop(0, CPW // GDEPTH, body, 0)
        plsc.subcore_barrier()
        pltpu.sync_copy(acc.at[pl.ds(s * rpt, rpt)],
                        out.at[c].at[pl.ds(s * rpt, rpt)])

    return segsum


def _segment_sum_sc(table, srcc, dstc, zrows):
    return _make_segsum(table.shape[1])(table, srcc, dstc, zrows)


def _proj_body(x_ref, w_ref, o_ref):
    o_ref[...] = jnp.dot(x_ref[...], w_ref[...],
                         preferred_element_type=jnp.float32)


def _proj(xp, w):
    n, k = xp.shape
    m = w.shape[1]
    return pl.pallas_call(
        _proj_body,
        grid=(n // ROW_BLK,),
        in_specs=[pl.BlockSpec((ROW_BLK, k), lambda i: (i, 0)),
                  pl.BlockSpec((k, m), lambda i: (0, 0))],
        out_specs=pl.BlockSpec((ROW_BLK, m), lambda i: (i, 0)),
        out_shape=jax.ShapeDtypeStruct((n, m), jnp.float32),
    )(xp, w)


def _mid_body(a0_ref, a1_ref, x_ref, wroot_ref, b1_ref, wl1_ref, bl1_ref,
              w2rel_ref, w2root_ref, b2_ref, p2_ref, r2_ref):
    h = (a0_ref[...] + a1_ref[...] + b1_ref[...]
         + jnp.dot(x_ref[...], wroot_ref[...],
                   preferred_element_type=jnp.float32))
    t = jax.nn.relu(jnp.dot(h, wl1_ref[...],
                            preferred_element_type=jnp.float32) + bl1_ref[...])
    p2_ref[...] = jnp.dot(t, w2rel_ref[...],
                          preferred_element_type=jnp.float32)
    r2_ref[...] = (jnp.dot(t, w2root_ref[...],
                           preferred_element_type=jnp.float32) + b2_ref[...])


def _mid(a0, a1, xp, w1_root, b1, wl1, bl1, w2_rel, w2_root, b2):
    n = xp.shape[0]
    full = lambda shape: pl.BlockSpec(shape, lambda i: (0, 0))
    row = lambda m: pl.BlockSpec((ROW_BLK, m), lambda i: (i, 0))
    return pl.pallas_call(
        _mid_body,
        grid=(n // ROW_BLK,),
        in_specs=[row(64), row(64), row(128), full((128, 64)), full((1, 64)),
                  full((64, 32)), full((1, 32)), full((32, 16)),
                  full((32, 16)), full((1, 16))],
        out_specs=[row(16), row(16)],
        out_shape=[jax.ShapeDtypeStruct((n, 16), jnp.float32),
                   jax.ShapeDtypeStruct((n, 16), jnp.float32)],
    )(a0, a1, xp, w1_root, b1.reshape(1, 64), wl1, bl1.reshape(1, 32),
      w2_rel, w2_root, b2.reshape(1, 16))


def _final_body(a0_ref, a1_ref, r2_ref, wl2_ref, bl2_ref, o_ref):
    h2 = a0_ref[...] + a1_ref[...] + r2_ref[...]
    logits = jnp.dot(h2, wl2_ref[...],
                     preferred_element_type=jnp.float32) + bl2_ref[...]
    m = jnp.max(logits, axis=1, keepdims=True)
    sh = logits - m
    lse = jnp.log(jnp.sum(jnp.exp(sh), axis=1, keepdims=True))
    o_ref[...] = sh - lse


def _final(a0, a1, r2, wl2, bl2):
    n = a0.shape[0]
    ncls = wl2.shape[1]
    full = lambda shape: pl.BlockSpec(shape, lambda i: (0, 0))
    row = lambda m: pl.BlockSpec((ROW_BLK, m), lambda i: (i, 0))
    return pl.pallas_call(
        _final_body,
        grid=(n // ROW_BLK,),
        in_specs=[row(16), row(16), row(16), full((16, ncls)),
                  full((1, ncls))],
        out_specs=row(ncls),
        out_shape=jax.ShapeDtypeStruct((n, ncls), jnp.float32),
    )(a0, a1, r2, wl2, bl2.reshape(1, ncls))


def kernel(x, edge_index, W1_rel, b1, W1_root, Wl1, bl1, W2_rel, b2, W2_root,
           Wl2, bl2):
    n = x.shape[0]
    e = edge_index.shape[1]
    xp = jnp.pad(x, ((0, N_PAD - n), (0, 0)))
    src = edge_index[0].astype(jnp.int32)
    dst = edge_index[1].astype(jnp.int32)
    pad_e = N_CHUNKS * CHUNK - e
    srcc = jnp.concatenate([src, jnp.full((pad_e,), n, jnp.int32)])
    srcc = srcc.reshape(N_CHUNKS, CHUNK)
    dstc = jnp.concatenate([dst, jnp.full((pad_e,), n, jnp.int32)])
    dstc = dstc.reshape(N_CHUNKS, CHUNK)
    z64 = jnp.zeros((N_PAD, 64), jnp.float32)
    z16 = jnp.zeros((N_PAD, 16), jnp.float32)

    p1 = _proj(xp, W1_rel)
    agg1 = _segment_sum_sc(p1, srcc, dstc, z64)
    p2, r2 = _mid(agg1[0], agg1[1], xp, W1_root, b1, Wl1, bl1,
                  W2_rel, W2_root, b2)
    agg2 = _segment_sum_sc(p2, srcc, dstc, z16)
    out = _final(agg2[0], agg2[1], r2, Wl2, bl2)
    return out[:n]


# speedup vs baseline: 18.1439x; 2.0809x over previous
import functools

import jax
import jax.numpy as jnp
from jax import lax
from jax.experimental import pallas as pl
from jax.experimental.pallas import tpu as pltpu
from jax.experimental.pallas import tpu_sc as plsc

N_NODES = 10000
N_EDGES = 320000
N_PAD = 10240
NC, NS = 2, 16
NW = NC * NS
CHUNK = 128
CPW = 80
N_CHUNKS = NW * CPW
ROW_BLK = 1280


GDEPTH = 8


@functools.lru_cache(maxsize=None)
def _make_segsum(d):
    rpt = N_PAD // NS
    mesh = plsc.VectorSubcoreMesh(core_axis_name="c", subcore_axis_name="s")

    @functools.partial(
        pl.kernel,
        out_type=jax.ShapeDtypeStruct((NC, N_PAD, d), jnp.float32),
        mesh=mesh,
        scratch_types=[
            pltpu.VMEM((CPW, CHUNK), jnp.int32),
            pltpu.VMEM((CPW, CHUNK), jnp.int32),
            pltpu.VMEM((GDEPTH, CHUNK, d), jnp.float32),
            pltpu.VMEM_SHARED((N_PAD, d), jnp.float32),
            pltpu.SemaphoreType.DMA((GDEPTH,)),
            pltpu.SemaphoreType.DMA,
        ],
        compiler_params=pltpu.CompilerParams(use_tc_tiling_on_sc=False),
    )
    def segsum(table, srcc, dstc, zrows, out, src_v, dst_v, rows_v, acc,
               gsem, sem):
        c = lax.axis_index("c")
        s = lax.axis_index("s")
        wid = c * NS + s
        cz = pltpu.async_copy(zrows.at[pl.ds(s * rpt, rpt)],
                              acc.at[pl.ds(s * rpt, rpt)], sem)
        ci = pltpu.async_copy(srcc.at[pl.ds(wid * CPW, CPW)], src_v, sem)
        cj = pltpu.async_copy(dstc.at[pl.ds(wid * CPW, CPW)], dst_v, sem)
        cz.wait()
        ci.wait()
        cj.wait()
        plsc.subcore_barrier()

        def gather(cc, b):
            pltpu.async_copy(table.at[src_v.at[cc]], rows_v.at[b], gsem.at[b])

        def gather_wait(cc, b):
            pltpu.make_async_copy(table.at[src_v.at[cc]], rows_v.at[b],
                                  gsem.at[b]).wait()

        for b in range(GDEPTH):
            gather(b, b)

        def body(i, carry):
            j = i * GDEPTH
            for b in range(GDEPTH):
                cc = j + b
                gather_wait(cc, b)
                pltpu.sync_copy(rows_v.at[b], acc.at[dst_v.at[cc]], add=True)
                nxt = cc + GDEPTH

                @pl.when(nxt < CPW)
                def _():
                    gather(nxt, b)
            return carry

        lax.fori_loop(0, CPW // GDEPTH, body, 0)
        plsc.subcore_barrier()
        pltpu.sync_copy(acc.at[pl.ds(s * rpt, rpt)],
                        out.at[c].at[pl.ds(s * rpt, rpt)])

    return segsum


def _segment_sum_sc(table, srcc, dstc, zrows):
    return _make_segsum(table.shape[1])(table, srcc, dstc, zrows)


def _proj_body(x_ref, w_ref, o_ref):
    o_ref[...] = jnp.dot(x_ref[...], w_ref[...],
                         preferred_element_type=jnp.float32)


def _proj(xp, w):
    n, k = xp.shape
    m = w.shape[1]
    return pl.pallas_call(
        _proj_body,
        grid=(n // ROW_BLK,),
        in_specs=[pl.BlockSpec((ROW_BLK, k), lambda i: (i, 0)),
                  pl.BlockSpec((k, m), lambda i: (0, 0))],
        out_specs=pl.BlockSpec((ROW_BLK, m), lambda i: (i, 0)),
        out_shape=jax.ShapeDtypeStruct((n, m), jnp.float32),
    )(xp, w)


def _mid_body(a0_ref, a1_ref, x_ref, wroot_ref, b1_ref, wl1_ref, bl1_ref,
              w2rel_ref, w2root_ref, b2_ref, p2_ref, r2_ref):
    h = (a0_ref[...] + a1_ref[...] + b1_ref[...]
         + jnp.dot(x_ref[...], wroot_ref[...],
                   preferred_element_type=jnp.float32))
    t = jax.nn.relu(jnp.dot(h, wl1_ref[...],
                            preferred_element_type=jnp.float32) + bl1_ref[...])
    p2_ref[...] = jnp.dot(t, w2rel_ref[...],
                          preferred_element_type=jnp.float32)
    r2_ref[...] = (jnp.dot(t, w2root_ref[...],
                           preferred_element_type=jnp.float32) + b2_ref[...])


def _mid(a0, a1, xp, w1_root, b1, wl1, bl1, w2_rel, w2_root, b2):
    n = xp.shape[0]
    full = lambda shape: pl.BlockSpec(shape, lambda i: (0, 0))
    row = lambda m: pl.BlockSpec((ROW_BLK, m), lambda i: (i, 0))
    return pl.pallas_call(
        _mid_body,
        grid=(n // ROW_BLK,),
        in_specs=[row(64), row(64), row(128), full((128, 64)), full((1, 64)),
                  full((64, 32)), full((1, 32)), full((32, 16)),
                  full((32, 16)), full((1, 16))],
        out_specs=[row(16), row(16)],
        out_shape=[jax.ShapeDtypeStruct((n, 16), jnp.float32),
                   jax.ShapeDtypeStruct((n, 16), jnp.float32)],
    )(a0, a1, xp, w1_root, b1.reshape(1, 64), wl1, bl1.reshape(1, 32),
      w2_rel, w2_root, b2.reshape(1, 16))


def _final_body(a0_ref, a1_ref, r2_ref, wl2_ref, bl2_ref, o_ref):
    h2 = a0_ref[...] + a1_ref[...] + r2_ref[...]
    logits = jnp.dot(h2, wl2_ref[...],
                     preferred_element_type=jnp.float32) + bl2_ref[...]
    m = jnp.max(logits, axis=1, keepdims=True)
    sh = logits - m
    lse = jnp.log(jnp.sum(jnp.exp(sh), axis=1, keepdims=True))
    o_ref[...] = sh - lse


def _final(a0, a1, r2, wl2, bl2):
    n = a0.shape[0]
    ncls = wl2.shape[1]
    full = lambda shape: pl.BlockSpec(shape, lambda i: (0, 0))
    row = lambda m: pl.BlockSpec((ROW_BLK, m), lambda i: (i, 0))
    return pl.pallas_call(
        _final_body,
        grid=(n // ROW_BLK,),
        in_specs=[row(16), row(16), row(16), full((16, ncls)),
                  full((1, ncls))],
        out_specs=row(ncls),
        out_shape=jax.ShapeDtypeStruct((n, ncls), jnp.float32),
    )(a0, a1, r2, wl2, bl2.reshape(1, ncls))


def kernel(x, edge_index, W1_rel, b1, W1_root, Wl1, bl1, W2_rel, b2, W2_root,
           Wl2, bl2):
    n = x.shape[0]
    e = edge_index.shape[1]
    xp = jnp.pad(x, ((0, N_PAD - n), (0, 0)))
    src = edge_index[0].astype(jnp.int32)
    dst = edge_index[1].astype(jnp.int32)
    pad_e = N_CHUNKS * CHUNK - e
    pad_idx = n + (jnp.arange(pad_e, dtype=jnp.int32) % (N_PAD - n))
    srcc = jnp.concatenate([src, pad_idx]).reshape(N_CHUNKS, CHUNK)
    dstc = jnp.concatenate([dst, pad_idx]).reshape(N_CHUNKS, CHUNK)
    z64 = jnp.zeros((N_PAD, 64), jnp.float32)
    z16 = jnp.zeros((N_PAD, 16), jnp.float32)

    p1 = _proj(xp, W1_rel)
    agg1 = _segment_sum_sc(p1, srcc, dstc, z64)
    p2, r2 = _mid(agg1[0], agg1[1], xp, W1_root, b1, Wl1, bl1,
                  W2_rel, W2_root, b2)
    agg2 = _segment_sum_sc(p2, srcc, dstc, z16)
    out = _final(agg2[0], agg2[1], r2, Wl2, bl2)
    return out[:n]


# speedup vs baseline: 19.7376x; 1.0878x over previous
import functools

import jax
import jax.numpy as jnp
from jax import lax
from jax.experimental import pallas as pl
from jax.experimental.pallas import tpu as pltpu
from jax.experimental.pallas import tpu_sc as plsc

N_NODES = 10000
N_EDGES = 320000
NC, NS = 2, 16
NW = NC * NS
CHUNK = 128
N_CHUNKS = N_EDGES // CHUNK
CPW = 80
CPW_LAST = N_CHUNKS - (NW - 1) * CPW
GDEPTH = 4
RPT = N_NODES // NS
ROW_BLK = 2000


@functools.lru_cache(maxsize=None)
def _make_segsum(d):
    mesh = plsc.VectorSubcoreMesh(core_axis_name="c", subcore_axis_name="s")

    @functools.partial(
        pl.kernel,
        out_type=jax.ShapeDtypeStruct((NC, N_NODES, d), jnp.float32),
        mesh=mesh,
        scratch_types=[
            pltpu.VMEM((CPW, CHUNK), jnp.int32),
            pltpu.VMEM((CPW, CHUNK), jnp.int32),
            pltpu.VMEM((GDEPTH, CHUNK, d), jnp.float32),
            pltpu.VMEM_SHARED((N_NODES, d), jnp.float32),
            pltpu.SemaphoreType.DMA((GDEPTH,)),
            pltpu.SemaphoreType.DMA,
        ],
        compiler_params=pltpu.CompilerParams(use_tc_tiling_on_sc=False),
    )
    def segsum(table, srcc, dstc, zrows, out, src_v, dst_v, rows_v, acc,
               gsem, sem):
        c = lax.axis_index("c")
        s = lax.axis_index("s")
        wid = c * NS + s
        last = wid == NW - 1
        nchunks = jnp.where(last, CPW_LAST, CPW)
        pltpu.sync_copy(zrows.at[pl.ds(s * RPT, RPT)],
                        acc.at[pl.ds(s * RPT, RPT)])

        @pl.when(jnp.logical_not(last))
        def _():
            pltpu.sync_copy(srcc.at[pl.ds(wid * CPW, CPW)], src_v)
            pltpu.sync_copy(dstc.at[pl.ds(wid * CPW, CPW)], dst_v)

        @pl.when(last)
        def _():
            pltpu.sync_copy(srcc.at[pl.ds(wid * CPW, CPW_LAST)],
                            src_v.at[pl.ds(0, CPW_LAST)])
            pltpu.sync_copy(dstc.at[pl.ds(wid * CPW, CPW_LAST)],
                            dst_v.at[pl.ds(0, CPW_LAST)])

        plsc.subcore_barrier()

        def gather(cc, b):
            pltpu.async_copy(table.at[src_v.at[cc]], rows_v.at[b], gsem.at[b])

        def gather_wait(cc, b):
            pltpu.make_async_copy(table.at[src_v.at[cc]], rows_v.at[b],
                                  gsem.at[b]).wait()

        for b in range(GDEPTH):
            gather(b, b)

        def body(i, carry):
            j = i * GDEPTH
            for b in range(GDEPTH):
                cc = j + b
                gather_wait(cc, b)
                pltpu.sync_copy(rows_v.at[b], acc.at[dst_v.at[cc]], add=True)
                nxt = cc + GDEPTH

                @pl.when(nxt < nchunks)
                def _():
                    gather(nxt, b)
            return carry

        lax.fori_loop(0, nchunks // GDEPTH, body, 0)
        plsc.subcore_barrier()
        pltpu.sync_copy(acc.at[pl.ds(s * RPT, RPT)],
                        out.at[c].at[pl.ds(s * RPT, RPT)])

    return segsum


def _segment_sum_sc(table, srcc, dstc, zrows):
    return _make_segsum(table.shape[1])(table, srcc, dstc, zrows)


def _proj_body(x_ref, w_ref, o_ref):
    o_ref[...] = jnp.dot(x_ref[...], w_ref[...],
                         preferred_element_type=jnp.float32)


def _proj(x, w):
    n, k = x.shape
    m = w.shape[1]
    return pl.pallas_call(
        _proj_body,
        grid=(n // ROW_BLK,),
        in_specs=[pl.BlockSpec((ROW_BLK, k), lambda i: (i, 0)),
                  pl.BlockSpec((k, m), lambda i: (0, 0))],
        out_specs=pl.BlockSpec((ROW_BLK, m), lambda i: (i, 0)),
        out_shape=jax.ShapeDtypeStruct((n, m), jnp.float32),
    )(x, w)


def _mid_body(agg_ref, x_ref, wroot_ref, b1_ref, wl1_ref, bl1_ref,
              w2rel_ref, w2root_ref, b2_ref, p2_ref, r2_ref):
    h = (agg_ref[0] + agg_ref[1] + b1_ref[...]
         + jnp.dot(x_ref[...], wroot_ref[...],
                   preferred_element_type=jnp.float32))
    t = jax.nn.relu(jnp.dot(h, wl1_ref[...],
                            preferred_element_type=jnp.float32) + bl1_ref[...])
    p2_ref[...] = jnp.dot(t, w2rel_ref[...],
                          preferred_element_type=jnp.float32)
    r2_ref[...] = (jnp.dot(t, w2root_ref[...],
                           preferred_element_type=jnp.float32) + b2_ref[...])


def _mid(agg1, x, w1_root, b1, wl1, bl1, w2_rel, w2_root, b2):
    n = x.shape[0]
    full = lambda shape: pl.BlockSpec(shape, lambda i: tuple(0 for _ in shape))
    row = lambda m: pl.BlockSpec((ROW_BLK, m), lambda i: (i, 0))
    return pl.pallas_call(
        _mid_body,
        grid=(n // ROW_BLK,),
        in_specs=[pl.BlockSpec((NC, ROW_BLK, 64), lambda i: (0, i, 0)),
                  row(128), full((128, 64)), full((1, 64)),
                  full((64, 32)), full((1, 32)), full((32, 16)),
                  full((32, 16)), full((1, 16))],
        out_specs=[row(16), row(16)],
        out_shape=[jax.ShapeDtypeStruct((n, 16), jnp.float32),
                   jax.ShapeDtypeStruct((n, 16), jnp.float32)],
    )(agg1, x, w1_root, b1.reshape(1, 64), wl1, bl1.reshape(1, 32),
      w2_rel, w2_root, b2.reshape(1, 16))


def _final_body(agg_ref, r2_ref, wl2_ref, bl2_ref, o_ref):
    h2 = agg_ref[0] + agg_ref[1] + r2_ref[...]
    logits = jnp.dot(h2, wl2_ref[...],
                     preferred_element_type=jnp.float32) + bl2_ref[...]
    m = jnp.max(logits, axis=1, keepdims=True)
    sh = logits - m
    lse = jnp.log(jnp.sum(jnp.exp(sh), axis=1, keepdims=True))
    o_ref[...] = sh - lse


def _final(agg2, r2, wl2, bl2):
    n = r2.shape[0]
    ncls = wl2.shape[1]
    full = lambda shape: pl.BlockSpec(shape, lambda i: tuple(0 for _ in shape))
    row = lambda m: pl.BlockSpec((ROW_BLK, m), lambda i: (i, 0))
    return pl.pallas_call(
        _final_body,
        grid=(n // ROW_BLK,),
        in_specs=[pl.BlockSpec((NC, ROW_BLK, 16), lambda i: (0, i, 0)),
                  row(16), full((16, ncls)), full((1, ncls))],
        out_specs=row(ncls),
        out_shape=jax.ShapeDtypeStruct((n, ncls), jnp.float32),
    )(agg2, r2, wl2, bl2.reshape(1, ncls))


def kernel(x, edge_index, W1_rel, b1, W1_root, Wl1, bl1, W2_rel, b2, W2_root,
           Wl2, bl2):
    srcc = edge_index[0].astype(jnp.int32).reshape(N_CHUNKS, CHUNK)
    dstc = edge_index[1].astype(jnp.int32).reshape(N_CHUNKS, CHUNK)
    z64 = jnp.zeros((N_NODES, 64), jnp.float32)
    z16 = jnp.zeros((N_NODES, 16), jnp.float32)

    p1 = _proj(x, W1_rel)
    agg1 = _segment_sum_sc(p1, srcc, dstc, z64)
    p2, r2 = _mid(agg1, x, W1_root, b1, Wl1, bl1,
                  W2_rel, W2_root, b2)
    agg2 = _segment_sum_sc(p2, srcc, dstc, z16)
    return _final(agg2, r2, Wl2, bl2)
